# baseline (speedup 1.0000x reference)
import jax
import jax.numpy as jnp
from jax import lax
from jax.experimental import pallas as pl
from jax.experimental.pallas import tpu as pltpu
from jax.experimental.pallas import tpu_sc as plsc

N = 20
IN_CH = 4
HID = 32
E = 380

NP = 32
EP = 384
EG = EP // 16
INVS = float(1.0 / (1.0 + 1e-5) ** 0.5)

GATES = (0, 1, 4, 5, 6, 7)

OFF_X = 0
OFF_EW = 96
OFF_W1 = OFF_EW + 384
OFF_W2 = OFF_W1 + 128
OFF_WL = OFF_W2 + 1024
OFF_BL = OFF_WL + 80
OFF_B1 = OFF_BL + 16
OFF_G1 = OFF_B1 + HID
OFF_BE1 = OFF_G1 + HID
OFF_B2 = OFF_BE1 + HID
OFF_G2 = OFF_B2 + HID
OFF_BE2 = OFF_G2 + HID
OFF_BIH1 = OFF_BE2 + HID
OFF_BHH1 = OFF_BIH1 + 128
OFF_BIH2 = OFF_BHH1 + 128
OFF_BHH2 = OFF_BIH2 + 128
PTOT = OFF_BHH2 + 128


def _sigmoid(v):
    return 1.0 / (1.0 + jnp.exp(-v))


def _tanh(v):
    return 2.0 / (1.0 + jnp.exp(-2.0 * v)) - 1.0


def _rsqrt(d):
    i = plsc.bitcast(d, jnp.int32)
    y = plsc.bitcast(jnp.int32(0x5F3759DF) - (i >> 1), jnp.float32)
    for _ in range(3):
        y = y * (1.5 - 0.5 * d * y * y)
    return y


def _body(pk_h, ei_h, wc_h,
          out_hbm,
          pv, evv, wcv,
          dv, av, xwv, h1v, obv, sem, semw):
    wid = lax.axis_index("s") * 2 + lax.axis_index("c")

    @pl.when(wid < N)
    def _():
        hw = pltpu.async_copy(wc_h, wcv, semw)
        handles = [pltpu.async_copy(pk_h, pv, sem),
                   pltpu.async_copy(ei_h, evv, sem)]

        zero = jnp.zeros((16,), jnp.float32)
        one = jnp.full((16,), 1.0, jnp.float32)
        lane = lax.iota(jnp.int32, 16)

        for i in range(NP * NP // 16):
            av[pl.ds(i * 16, 16)] = zero
        dv[pl.ds(0, 16)] = zero
        dv[pl.ds(16, 16)] = zero

        for h in handles:
            h.wait()

        for g in range(EG):
            cvec = evv[pl.ds(EP + g * 16, 16)]
            wvec = pv[pl.ds(OFF_EW + g * 16, 16)]
            plsc.addupdate_scatter(dv, [cvec], wvec)

        dv[pl.ds(0, 16)] = dv[pl.ds(0, 16)] + one
        dv[pl.ds(16, 16)] = dv[pl.ds(16, 16)] + jnp.where(lane < (N - 16), 1.0, 0.0)

        for ch in range(2):
            d = dv[pl.ds(ch * 16, 16)]
            dv[pl.ds(ch * 16, 16)] = jnp.where(d > 0, _rsqrt(d), 0.0)

        for g in range(EG):
            rvec = evv[pl.ds(g * 16, 16)]
            cvec = evv[pl.ds(EP + g * 16, 16)]
            wvec = pv[pl.ds(OFF_EW + g * 16, 16)]
            dr = plsc.load_gather(dv, [rvec])
            dc = plsc.load_gather(dv, [cvec])
            plsc.addupdate_scatter(av, [cvec * NP + rvec], dr * wvec * dc)

        d0 = dv[pl.ds(0, 16)]
        plsc.addupdate_scatter(av, [lane * (NP + 1)], d0 * d0)
        d1 = dv[pl.ds(16, 16)]
        plsc.addupdate_scatter(av, [(lane + 16) * (NP + 1)], d1 * d1)

        def xmat(src_ref, src_stride, src_off, k_dim, w_off, dst_ref):
            def nb(bi, c):
                n = bi * 2
                accs = [zero] * 4
                sv = [[src_ref[pl.ds(src_off + (n + j) * src_stride + q * 16, 16)]
                       for q in range((k_dim + 15) // 16)] for j in range(2)]
                for k in range(k_dim):
                    wr0 = pv[pl.ds(w_off + k * HID, 16)]
                    wr1 = pv[pl.ds(w_off + k * HID + 16, 16)]
                    for j in range(2):
                        s = sv[j][k // 16][k % 16]
                        accs[2 * j] = accs[2 * j] + s * wr0
                        accs[2 * j + 1] = accs[2 * j + 1] + s * wr1
                for j in range(2):
                    dst_ref[pl.ds((n + j) * HID, 16)] = accs[2 * j]
                    dst_ref[pl.ds((n + j) * HID + 16, 16)] = accs[2 * j + 1]
                return c
            lax.fori_loop(0, N // 2, nb, 0)

        def aggregate(b_off, g_off, be_off, dst_ref):
            gm0 = pv[pl.ds(g_off, 16)] * INVS
            gm1 = pv[pl.ds(g_off + 16, 16)] * INVS
            bt0 = pv[pl.ds(be_off, 16)]
            bt1 = pv[pl.ds(be_off + 16, 16)]
            ib0 = pv[pl.ds(b_off, 16)]
            ib1 = pv[pl.ds(b_off + 16, 16)]
            def cb(bi, c):
                cc = bi * 2
                accs = [ib0, ib1, ib0, ib1]
                ar = [[av[pl.ds((cc + j) * NP, 16)],
                       av[pl.ds((cc + j) * NP + 16, 16)]] for j in range(2)]
                for r in range(N):
                    x0 = xwv[pl.ds(r * HID, 16)]
                    x1 = xwv[pl.ds(r * HID + 16, 16)]
                    for j in range(2):
                        s = ar[j][r // 16][r % 16]
                        accs[2 * j] = accs[2 * j] + s * x0
                        accs[2 * j + 1] = accs[2 * j + 1] + s * x1
                for j in range(2):
                    dst_ref[pl.ds((cc + j) * HID, 16)] = (
                        jnp.maximum(accs[2 * j], 0.0) * gm0 + bt0)
                    dst_ref[pl.ds((cc + j) * HID + 16, 16)] = (
                        jnp.maximum(accs[2 * j + 1], 0.0) * gm1 + bt1)
                return c
            lax.fori_loop(0, N // 2, cb, 0)

        xmat(pv, IN_CH, OFF_X, IN_CH, OFF_W1, xwv)
        aggregate(OFF_B1, OFF_G1, OFF_BE1, h1v)

        n = wid
        ar0 = av[pl.ds(n * NP, 16)]
        ar1 = av[pl.ds(n * NP + 16, 16)]
        y0 = zero
        y1 = zero
        for r in range(N):
            s = (ar0 if r < 16 else ar1)[r % 16]
            y0 = y0 + s * h1v[pl.ds(r * HID, 16)]
            y1 = y1 + s * h1v[pl.ds(r * HID + 16, 16)]
        a20 = pv[pl.ds(OFF_B2, 16)]
        a21 = pv[pl.ds(OFF_B2 + 16, 16)]
        for k in range(HID):
            s = (y0 if k < 16 else y1)[k % 16]
            a20 = a20 + s * pv[pl.ds(OFF_W2 + k * HID, 16)]
            a21 = a21 + s * pv[pl.ds(OFF_W2 + k * HID + 16, 16)]
        h2a = jnp.maximum(a20, 0.0) * (pv[pl.ds(OFF_G2, 16)] * INVS) \
            + pv[pl.ds(OFF_BE2, 16)]
        h2b = jnp.maximum(a21, 0.0) * (pv[pl.ds(OFF_G2 + 16, 16)] * INVS) \
            + pv[pl.ds(OFF_BE2 + 16, 16)]

        def lstm(srcs, w_off, bih_off, bhh_off):
            acc = [pv[pl.ds(bih_off + ch * 16, 16)] + pv[pl.ds(bhh_off + ch * 16, 16)]
                   for ch in GATES]
            for q, svec in enumerate(srcs):
                for k16 in range(16):
                    s = svec[k16]
                    k = q * 16 + k16
                    for j, ch in enumerate(GATES):
                        acc[j] = acc[j] + s * wcv[pl.ds(w_off + k * 128 + ch * 16, 16)]
            i0, i1 = _sigmoid(acc[0]), _sigmoid(acc[1])
            g0, g1 = _tanh(acc[2]), _tanh(acc[3])
            o0, o1 = _sigmoid(acc[4]), _sigmoid(acc[5])
            return o0 * _tanh(i0 * g0), o1 * _tanh(i1 * g1)

        hw.wait()
        h1a = h1v[pl.ds(n * HID, 16)]
        h1b = h1v[pl.ds(n * HID + 16, 16)]
        H1a, H1b = lstm([h1a, h1b, h2a, h2b], 0, OFF_BIH1, OFF_BHH1)
        H2a, H2b = lstm([H1a, H1b], 64 * 128, OFF_BIH2, OFF_BHH2)

        xrow = pv[pl.ds(OFF_X + n * IN_CH, 16)]
        v = (jnp.maximum(H1a, 0.0) * pv[pl.ds(OFF_WL, 16)]
             + jnp.maximum(H1b, 0.0) * pv[pl.ds(OFF_WL + 16, 16)]
             + jnp.maximum(H2a, 0.0) * pv[pl.ds(OFF_WL + 32, 16)]
             + jnp.maximum(H2b, 0.0) * pv[pl.ds(OFF_WL + 48, 16)]
             + jnp.where(lane < IN_CH,
                         jnp.maximum(xrow, 0.0) * pv[pl.ds(OFF_WL + 64, 16)], 0.0))
        tot = jnp.sum(v) + pv[pl.ds(OFF_BL, 16)][0]
        obv[...] = jnp.full((16,), 0.0, jnp.float32) + tot
        pltpu.sync_copy(obv, out_hbm.at[n])


@jax.jit
def kernel(x, edge_index, edge_weight, W1, b1, gamma1, beta1, W2, b2, gamma2,
           beta2, W_ih1, W_hh1, b_ih1, b_hh1, W_ih2, W_hh2, b_ih2, b_hh2, Wl, bl):
    f32 = jnp.float32
    edges = jnp.pad(edge_index.astype(jnp.int32), ((0, 0), (0, EP - E))).reshape(-1)
    wi1t = W_ih1.astype(f32).T.reshape(-1)
    wi2t = W_ih2.astype(f32).T.reshape(-1)
    z16 = jnp.zeros((16,), f32)
    wcat = jnp.concatenate([wi1t, wi2t])
    pack = jnp.concatenate([
        x.astype(f32).reshape(-1), z16,
        edge_weight.astype(f32), jnp.zeros((EP - E,), f32),
        W1.astype(f32).reshape(-1),
        W2.astype(f32).reshape(-1),
        Wl.astype(f32).reshape(-1), jnp.zeros((12,), f32),
        bl.astype(f32), jnp.zeros((15,), f32),
        b1.astype(f32), gamma1.astype(f32), beta1.astype(f32),
        b2.astype(f32), gamma2.astype(f32), beta2.astype(f32),
        b_ih1.astype(f32), b_hh1.astype(f32),
        b_ih2.astype(f32), b_hh2.astype(f32),
    ])

    mesh = plsc.VectorSubcoreMesh(core_axis_name="c", subcore_axis_name="s")
    out = pl.kernel(
        _body,
        out_type=jax.ShapeDtypeStruct((N, 16), f32),
        mesh=mesh,
        compiler_params=pltpu.CompilerParams(needs_layout_passes=False),
        scratch_types=[
            pltpu.VMEM((PTOT,), f32),
            pltpu.VMEM((2 * EP,), jnp.int32),
            pltpu.VMEM((96 * 128,), f32),
            pltpu.VMEM((NP,), f32),
            pltpu.VMEM((NP * NP,), f32),
            pltpu.VMEM((N * HID,), f32),
            pltpu.VMEM((N * HID,), f32),
            pltpu.VMEM((16,), f32),
            pltpu.SemaphoreType.DMA,
            pltpu.SemaphoreType.DMA,
        ],
    )(pack, edges, wcat)
    return out[:, :1]


# speedup vs baseline: 1.0192x; 1.0192x over previous
import jax
import jax.numpy as jnp
from jax import lax
from jax.experimental import pallas as pl
from jax.experimental.pallas import tpu as pltpu
from jax.experimental.pallas import tpu_sc as plsc

N = 20
IN_CH = 4
HID = 32
E = 380

NP = 32
EP = 384
EG = EP // 16
INVS = float(1.0 / (1.0 + 1e-5) ** 0.5)

GATES = (0, 1, 4, 5, 6, 7)

OFF_X = 0
OFF_EW = 96
OFF_W1 = OFF_EW + 384
OFF_W2 = OFF_W1 + 128
OFF_WL = OFF_W2 + 1024
OFF_BL = OFF_WL + 80
OFF_B1 = OFF_BL + 16
OFF_G1 = OFF_B1 + HID
OFF_BE1 = OFF_G1 + HID
OFF_B2 = OFF_BE1 + HID
OFF_G2 = OFF_B2 + HID
OFF_BE2 = OFF_G2 + HID
OFF_BIH1 = OFF_BE2 + HID
OFF_BHH1 = OFF_BIH1 + 128
OFF_BIH2 = OFF_BHH1 + 128
OFF_BHH2 = OFF_BIH2 + 128
PTOT = OFF_BHH2 + 128


def _sigmoid(v):
    return 1.0 / (1.0 + jnp.exp(-v))


def _tanh(v):
    return 2.0 / (1.0 + jnp.exp(-2.0 * v)) - 1.0


def _rsqrt(d):
    i = plsc.bitcast(d, jnp.int32)
    y = plsc.bitcast(jnp.int32(0x5F3759DF) - (i >> 1), jnp.float32)
    for _ in range(3):
        y = y * (1.5 - 0.5 * d * y * y)
    return y


def _body(pk_h, ei_h, wc_h,
          out_hbm,
          pv, evv, wcv,
          dv, av, xwv, h1v, obv, sem, semw):
    wid = lax.axis_index("s") * 2 + lax.axis_index("c")

    @pl.when(wid < N)
    def _():
        hw = pltpu.async_copy(wc_h, wcv, semw)
        handles = [pltpu.async_copy(pk_h, pv, sem),
                   pltpu.async_copy(ei_h, evv, sem)]

        zero = jnp.zeros((16,), jnp.float32)
        one = jnp.full((16,), 1.0, jnp.float32)
        lane = lax.iota(jnp.int32, 16)

        for i in range(NP * NP // 16):
            av[pl.ds(i * 16, 16)] = zero
        dv[pl.ds(0, 16)] = zero
        dv[pl.ds(16, 16)] = zero

        for h in handles:
            h.wait()

        def degb(g, c):
            cvec = evv[pl.ds(EP + g * 16, 16)]
            wvec = pv[pl.ds(OFF_EW + g * 16, 16)]
            plsc.addupdate_scatter(dv, [cvec], wvec)
            return c
        lax.fori_loop(0, EG, degb, 0)

        dv[pl.ds(0, 16)] = dv[pl.ds(0, 16)] + one
        dv[pl.ds(16, 16)] = dv[pl.ds(16, 16)] + jnp.where(lane < (N - 16), 1.0, 0.0)

        for ch in range(2):
            d = dv[pl.ds(ch * 16, 16)]
            dv[pl.ds(ch * 16, 16)] = jnp.where(d > 0, _rsqrt(d), 0.0)

        def adjb(g, c):
            rvec = evv[pl.ds(g * 16, 16)]
            cvec = evv[pl.ds(EP + g * 16, 16)]
            wvec = pv[pl.ds(OFF_EW + g * 16, 16)]
            dr = plsc.load_gather(dv, [rvec])
            dc = plsc.load_gather(dv, [cvec])
            plsc.addupdate_scatter(av, [cvec * NP + rvec], dr * wvec * dc)
            return c
        lax.fori_loop(0, EG, adjb, 0)

        d0 = dv[pl.ds(0, 16)]
        plsc.addupdate_scatter(av, [lane * (NP + 1)], d0 * d0)
        d1 = dv[pl.ds(16, 16)]
        plsc.addupdate_scatter(av, [(lane + 16) * (NP + 1)], d1 * d1)

        def xmat(src_ref, src_stride, src_off, k_dim, w_off, dst_ref):
            def nb(bi, c):
                n = bi * 2
                accs = [zero] * 4
                sv = [[src_ref[pl.ds(src_off + (n + j) * src_stride + q * 16, 16)]
                       for q in range((k_dim + 15) // 16)] for j in range(2)]
                for k in range(k_dim):
                    wr0 = pv[pl.ds(w_off + k * HID, 16)]
                    wr1 = pv[pl.ds(w_off + k * HID + 16, 16)]
                    for j in range(2):
                        s = sv[j][k // 16][k % 16]
                        accs[2 * j] = accs[2 * j] + s * wr0
                        accs[2 * j + 1] = accs[2 * j + 1] + s * wr1
                for j in range(2):
                    dst_ref[pl.ds((n + j) * HID, 16)] = accs[2 * j]
                    dst_ref[pl.ds((n + j) * HID + 16, 16)] = accs[2 * j + 1]
                return c
            lax.fori_loop(0, N // 2, nb, 0)

        def aggregate(b_off, g_off, be_off, dst_ref):
            gm0 = pv[pl.ds(g_off, 16)] * INVS
            gm1 = pv[pl.ds(g_off + 16, 16)] * INVS
            bt0 = pv[pl.ds(be_off, 16)]
            bt1 = pv[pl.ds(be_off + 16, 16)]
            ib0 = pv[pl.ds(b_off, 16)]
            ib1 = pv[pl.ds(b_off + 16, 16)]
            def cb(bi, c):
                cc = bi * 2
                accs = [ib0, ib1, ib0, ib1]
                ar = [[av[pl.ds((cc + j) * NP, 16)],
                       av[pl.ds((cc + j) * NP + 16, 16)]] for j in range(2)]
                for r in range(N):
                    x0 = xwv[pl.ds(r * HID, 16)]
                    x1 = xwv[pl.ds(r * HID + 16, 16)]
                    for j in range(2):
                        s = ar[j][r // 16][r % 16]
                        accs[2 * j] = accs[2 * j] + s * x0
                        accs[2 * j + 1] = accs[2 * j + 1] + s * x1
                for j in range(2):
                    dst_ref[pl.ds((cc + j) * HID, 16)] = (
                        jnp.maximum(accs[2 * j], 0.0) * gm0 + bt0)
                    dst_ref[pl.ds((cc + j) * HID + 16, 16)] = (
                        jnp.maximum(accs[2 * j + 1], 0.0) * gm1 + bt1)
                return c
            lax.fori_loop(0, N // 2, cb, 0)

        xmat(pv, IN_CH, OFF_X, IN_CH, OFF_W1, xwv)
        aggregate(OFF_B1, OFF_G1, OFF_BE1, h1v)

        n = wid
        ar0 = av[pl.ds(n * NP, 16)]
        ar1 = av[pl.ds(n * NP + 16, 16)]
        y0 = zero
        y1 = zero
        for r in range(N):
            s = (ar0 if r < 16 else ar1)[r % 16]
            y0 = y0 + s * h1v[pl.ds(r * HID, 16)]
            y1 = y1 + s * h1v[pl.ds(r * HID + 16, 16)]
        a20 = pv[pl.ds(OFF_B2, 16)]
        a21 = pv[pl.ds(OFF_B2 + 16, 16)]
        for k in range(HID):
            s = (y0 if k < 16 else y1)[k % 16]
            a20 = a20 + s * pv[pl.ds(OFF_W2 + k * HID, 16)]
            a21 = a21 + s * pv[pl.ds(OFF_W2 + k * HID + 16, 16)]
        h2a = jnp.maximum(a20, 0.0) * (pv[pl.ds(OFF_G2, 16)] * INVS) \
            + pv[pl.ds(OFF_BE2, 16)]
        h2b = jnp.maximum(a21, 0.0) * (pv[pl.ds(OFF_G2 + 16, 16)] * INVS) \
            + pv[pl.ds(OFF_BE2 + 16, 16)]

        def lstm(srcs, w_off, bih_off, bhh_off):
            acc = [pv[pl.ds(bih_off + ch * 16, 16)] + pv[pl.ds(bhh_off + ch * 16, 16)]
                   for ch in GATES]
            for q, svec in enumerate(srcs):
                for k16 in range(16):
                    s = svec[k16]
                    k = q * 16 + k16
                    for j, ch in enumerate(GATES):
                        acc[j] = acc[j] + s * wcv[pl.ds(w_off + k * 128 + ch * 16, 16)]
            i0, i1 = _sigmoid(acc[0]), _sigmoid(acc[1])
            g0, g1 = _tanh(acc[2]), _tanh(acc[3])
            o0, o1 = _sigmoid(acc[4]), _sigmoid(acc[5])
            return o0 * _tanh(i0 * g0), o1 * _tanh(i1 * g1)

        hw.wait()
        h1a = h1v[pl.ds(n * HID, 16)]
        h1b = h1v[pl.ds(n * HID + 16, 16)]
        H1a, H1b = lstm([h1a, h1b, h2a, h2b], 0, OFF_BIH1, OFF_BHH1)
        H2a, H2b = lstm([H1a, H1b], 64 * 128, OFF_BIH2, OFF_BHH2)

        xrow = pv[pl.ds(OFF_X + n * IN_CH, 16)]
        v = (jnp.maximum(H1a, 0.0) * pv[pl.ds(OFF_WL, 16)]
             + jnp.maximum(H1b, 0.0) * pv[pl.ds(OFF_WL + 16, 16)]
             + jnp.maximum(H2a, 0.0) * pv[pl.ds(OFF_WL + 32, 16)]
             + jnp.maximum(H2b, 0.0) * pv[pl.ds(OFF_WL + 48, 16)]
             + jnp.where(lane < IN_CH,
                         jnp.maximum(xrow, 0.0) * pv[pl.ds(OFF_WL + 64, 16)], 0.0))
        tot = jnp.sum(v) + pv[pl.ds(OFF_BL, 16)][0]
        obv[...] = jnp.full((16,), 0.0, jnp.float32) + tot
        pltpu.sync_copy(obv, out_hbm.at[n])


@jax.jit
def kernel(x, edge_index, edge_weight, W1, b1, gamma1, beta1, W2, b2, gamma2,
           beta2, W_ih1, W_hh1, b_ih1, b_hh1, W_ih2, W_hh2, b_ih2, b_hh2, Wl, bl):
    f32 = jnp.float32
    edges = jnp.pad(edge_index.astype(jnp.int32), ((0, 0), (0, EP - E))).reshape(-1)
    wi1t = W_ih1.astype(f32).T.reshape(-1)
    wi2t = W_ih2.astype(f32).T.reshape(-1)
    z16 = jnp.zeros((16,), f32)
    wcat = jnp.concatenate([wi1t, wi2t])
    pack = jnp.concatenate([
        x.astype(f32).reshape(-1), z16,
        edge_weight.astype(f32), jnp.zeros((EP - E,), f32),
        W1.astype(f32).reshape(-1),
        W2.astype(f32).reshape(-1),
        Wl.astype(f32).reshape(-1), jnp.zeros((12,), f32),
        bl.astype(f32), jnp.zeros((15,), f32),
        b1.astype(f32), gamma1.astype(f32), beta1.astype(f32),
        b2.astype(f32), gamma2.astype(f32), beta2.astype(f32),
        b_ih1.astype(f32), b_hh1.astype(f32),
        b_ih2.astype(f32), b_hh2.astype(f32),
    ])

    mesh = plsc.VectorSubcoreMesh(core_axis_name="c", subcore_axis_name="s")
    out = pl.kernel(
        _body,
        out_type=jax.ShapeDtypeStruct((N, 16), f32),
        mesh=mesh,
        compiler_params=pltpu.CompilerParams(needs_layout_passes=False),
        scratch_types=[
            pltpu.VMEM((PTOT,), f32),
            pltpu.VMEM((2 * EP,), jnp.int32),
            pltpu.VMEM((96 * 128,), f32),
            pltpu.VMEM((NP,), f32),
            pltpu.VMEM((NP * NP,), f32),
            pltpu.VMEM((N * HID,), f32),
            pltpu.VMEM((N * HID,), f32),
            pltpu.VMEM((16,), f32),
            pltpu.SemaphoreType.DMA,
            pltpu.SemaphoreType.DMA,
        ],
    )(pack, edges, wcat)
    return out[:, :1]


# speedup vs baseline: 1.0479x; 1.0282x over previous
import jax
import jax.numpy as jnp
from jax import lax
from jax.experimental import pallas as pl
from jax.experimental.pallas import tpu as pltpu
from jax.experimental.pallas import tpu_sc as plsc

N = 20
IN_CH = 4
HID = 32
E = 380

NP = 32
EP = 384
EG = EP // 16
INVS = float(1.0 / (1.0 + 1e-5) ** 0.5)

GATES = (0, 1, 4, 5, 6, 7)

OFF_X = 0
OFF_EW = 96
OFF_W1 = OFF_EW + 384
OFF_W2 = OFF_W1 + 128
OFF_WL = OFF_W2 + 1024
OFF_BL = OFF_WL + 80
OFF_B1 = OFF_BL + 16
OFF_G1 = OFF_B1 + HID
OFF_BE1 = OFF_G1 + HID
OFF_B2 = OFF_BE1 + HID
OFF_G2 = OFF_B2 + HID
OFF_BE2 = OFF_G2 + HID
OFF_BIH1 = OFF_BE2 + HID
OFF_BHH1 = OFF_BIH1 + 128
OFF_BIH2 = OFF_BHH1 + 128
OFF_BHH2 = OFF_BIH2 + 128
PTOT = OFF_BHH2 + 128


def _sigmoid(v):
    return 1.0 / (1.0 + jnp.exp(-v))


def _tanh(v):
    return 2.0 / (1.0 + jnp.exp(-2.0 * v)) - 1.0


def _bcast(v, k):
    idx = jnp.full((16, 1), 0, jnp.int32) + k
    return lax.gather(
        v, idx,
        dimension_numbers=lax.GatherDimensionNumbers(
            offset_dims=(), collapsed_slice_dims=(0,), start_index_map=(0,)),
        slice_sizes=(1,), mode=lax.GatherScatterMode.PROMISE_IN_BOUNDS)


def _rsqrt(d):
    i = plsc.bitcast(d, jnp.int32)
    y = plsc.bitcast(jnp.int32(0x5F3759DF) - (i >> 1), jnp.float32)
    for _ in range(3):
        y = y * (1.5 - 0.5 * d * y * y)
    return y


def _body(pk_h, ei_h, wc_h,
          out_hbm,
          pv, evv, wcv,
          dv, av, xwv, h1v, obv, sem, semw):
    wid = lax.axis_index("s") * 2 + lax.axis_index("c")

    @pl.when(wid < N)
    def _():
        hw = pltpu.async_copy(wc_h, wcv, semw)
        handles = [pltpu.async_copy(pk_h, pv, sem),
                   pltpu.async_copy(ei_h, evv, sem)]

        zero = jnp.zeros((16,), jnp.float32)
        one = jnp.full((16,), 1.0, jnp.float32)
        lane = lax.iota(jnp.int32, 16)

        for i in range(NP * NP // 16):
            av[pl.ds(i * 16, 16)] = zero
        dv[pl.ds(0, 16)] = zero
        dv[pl.ds(16, 16)] = zero

        for h in handles:
            h.wait()

        def degb(g, c):
            cvec = evv[pl.ds(EP + g * 16, 16)]
            wvec = pv[pl.ds(OFF_EW + g * 16, 16)]
            plsc.addupdate_scatter(dv, [cvec], wvec)
            return c
        lax.fori_loop(0, EG, degb, 0)

        dv[pl.ds(0, 16)] = dv[pl.ds(0, 16)] + one
        dv[pl.ds(16, 16)] = dv[pl.ds(16, 16)] + jnp.where(lane < (N - 16), 1.0, 0.0)

        for ch in range(2):
            d = dv[pl.ds(ch * 16, 16)]
            dv[pl.ds(ch * 16, 16)] = jnp.where(d > 0, _rsqrt(d), 0.0)

        def adjb(g, c):
            rvec = evv[pl.ds(g * 16, 16)]
            cvec = evv[pl.ds(EP + g * 16, 16)]
            wvec = pv[pl.ds(OFF_EW + g * 16, 16)]
            dr = plsc.load_gather(dv, [rvec])
            dc = plsc.load_gather(dv, [cvec])
            plsc.addupdate_scatter(av, [cvec * NP + rvec], dr * wvec * dc)
            return c
        lax.fori_loop(0, EG, adjb, 0)

        d0 = dv[pl.ds(0, 16)]
        plsc.addupdate_scatter(av, [lane * (NP + 1)], d0 * d0)
        d1 = dv[pl.ds(16, 16)]
        plsc.addupdate_scatter(av, [(lane + 16) * (NP + 1)], d1 * d1)

        def xmat(src_ref, src_stride, src_off, k_dim, w_off, dst_ref):
            def nb(bi, c):
                n = bi * 2
                accs = [zero] * 4
                sv = [[src_ref[pl.ds(src_off + (n + j) * src_stride + q * 16, 16)]
                       for q in range((k_dim + 15) // 16)] for j in range(2)]
                for k in range(k_dim):
                    wr0 = pv[pl.ds(w_off + k * HID, 16)]
                    wr1 = pv[pl.ds(w_off + k * HID + 16, 16)]
                    for j in range(2):
                        s = sv[j][k // 16][k % 16]
                        accs[2 * j] = accs[2 * j] + s * wr0
                        accs[2 * j + 1] = accs[2 * j + 1] + s * wr1
                for j in range(2):
                    dst_ref[pl.ds((n + j) * HID, 16)] = accs[2 * j]
                    dst_ref[pl.ds((n + j) * HID + 16, 16)] = accs[2 * j + 1]
                return c
            lax.fori_loop(0, N // 2, nb, 0)

        def aggregate(b_off, g_off, be_off, dst_ref):
            gm0 = pv[pl.ds(g_off, 16)] * INVS
            gm1 = pv[pl.ds(g_off + 16, 16)] * INVS
            bt0 = pv[pl.ds(be_off, 16)]
            bt1 = pv[pl.ds(be_off + 16, 16)]
            ib0 = pv[pl.ds(b_off, 16)]
            ib1 = pv[pl.ds(b_off + 16, 16)]
            def cb(bi, c):
                cc = bi * 2
                accs = [ib0, ib1, ib0, ib1]
                ar = [[av[pl.ds((cc + j) * NP, 16)],
                       av[pl.ds((cc + j) * NP + 16, 16)]] for j in range(2)]
                for r in range(N):
                    x0 = xwv[pl.ds(r * HID, 16)]
                    x1 = xwv[pl.ds(r * HID + 16, 16)]
                    for j in range(2):
                        s = ar[j][r // 16][r % 16]
                        accs[2 * j] = accs[2 * j] + s * x0
                        accs[2 * j + 1] = accs[2 * j + 1] + s * x1
                for j in range(2):
                    dst_ref[pl.ds((cc + j) * HID, 16)] = (
                        jnp.maximum(accs[2 * j], 0.0) * gm0 + bt0)
                    dst_ref[pl.ds((cc + j) * HID + 16, 16)] = (
                        jnp.maximum(accs[2 * j + 1], 0.0) * gm1 + bt1)
                return c
            lax.fori_loop(0, N // 2, cb, 0)

        xmat(pv, IN_CH, OFF_X, IN_CH, OFF_W1, xwv)
        aggregate(OFF_B1, OFF_G1, OFF_BE1, h1v)

        n = wid
        ar0 = av[pl.ds(n * NP, 16)]
        ar1 = av[pl.ds(n * NP + 16, 16)]
        y0 = zero
        y1 = zero
        for r in range(N):
            s = (ar0 if r < 16 else ar1)[r % 16]
            y0 = y0 + s * h1v[pl.ds(r * HID, 16)]
            y1 = y1 + s * h1v[pl.ds(r * HID + 16, 16)]
        a20 = pv[pl.ds(OFF_B2, 16)]
        a21 = pv[pl.ds(OFF_B2 + 16, 16)]
        for k in range(HID):
            s = (y0 if k < 16 else y1)[k % 16]
            a20 = a20 + s * pv[pl.ds(OFF_W2 + k * HID, 16)]
            a21 = a21 + s * pv[pl.ds(OFF_W2 + k * HID + 16, 16)]
        h2a = jnp.maximum(a20, 0.0) * (pv[pl.ds(OFF_G2, 16)] * INVS) \
            + pv[pl.ds(OFF_BE2, 16)]
        h2b = jnp.maximum(a21, 0.0) * (pv[pl.ds(OFF_G2 + 16, 16)] * INVS) \
            + pv[pl.ds(OFF_BE2 + 16, 16)]

        def lstm(srcs, w_off, bih_off, bhh_off):
            acc0 = tuple(
                pv[pl.ds(bih_off + ch * 16, 16)] + pv[pl.ds(bhh_off + ch * 16, 16)]
                for ch in GATES)
            def kb(ki, carry):
                acc = list(carry)
                for q, svec in enumerate(srcs):
                    sv = _bcast(svec, ki)
                    for j, ch in enumerate(GATES):
                        acc[j] = acc[j] + sv * wcv[
                            pl.ds(w_off + q * 2048 + ki * 128 + ch * 16, 16)]
                return tuple(acc)
            acc = list(lax.fori_loop(0, 16, kb, acc0))
            i0, i1 = _sigmoid(acc[0]), _sigmoid(acc[1])
            g0, g1 = _tanh(acc[2]), _tanh(acc[3])
            o0, o1 = _sigmoid(acc[4]), _sigmoid(acc[5])
            return o0 * _tanh(i0 * g0), o1 * _tanh(i1 * g1)

        hw.wait()
        h1a = h1v[pl.ds(n * HID, 16)]
        h1b = h1v[pl.ds(n * HID + 16, 16)]
        H1a, H1b = lstm([h1a, h1b, h2a, h2b], 0, OFF_BIH1, OFF_BHH1)
        H2a, H2b = lstm([H1a, H1b], 64 * 128, OFF_BIH2, OFF_BHH2)

        xrow = pv[pl.ds(OFF_X + n * IN_CH, 16)]
        v = (jnp.maximum(H1a, 0.0) * pv[pl.ds(OFF_WL, 16)]
             + jnp.maximum(H1b, 0.0) * pv[pl.ds(OFF_WL + 16, 16)]
             + jnp.maximum(H2a, 0.0) * pv[pl.ds(OFF_WL + 32, 16)]
             + jnp.maximum(H2b, 0.0) * pv[pl.ds(OFF_WL + 48, 16)]
             + jnp.where(lane < IN_CH,
                         jnp.maximum(xrow, 0.0) * pv[pl.ds(OFF_WL + 64, 16)], 0.0))
        tot = jnp.sum(v) + pv[pl.ds(OFF_BL, 16)][0]
        obv[...] = jnp.full((16,), 0.0, jnp.float32) + tot
        pltpu.sync_copy(obv, out_hbm.at[n])


@jax.jit
def kernel(x, edge_index, edge_weight, W1, b1, gamma1, beta1, W2, b2, gamma2,
           beta2, W_ih1, W_hh1, b_ih1, b_hh1, W_ih2, W_hh2, b_ih2, b_hh2, Wl, bl):
    f32 = jnp.float32
    edges = jnp.pad(edge_index.astype(jnp.int32), ((0, 0), (0, EP - E))).reshape(-1)
    wi1t = W_ih1.astype(f32).T.reshape(-1)
    wi2t = W_ih2.astype(f32).T.reshape(-1)
    z16 = jnp.zeros((16,), f32)
    wcat = jnp.concatenate([wi1t, wi2t])
    pack = jnp.concatenate([
        x.astype(f32).reshape(-1), z16,
        edge_weight.astype(f32), jnp.zeros((EP - E,), f32),
        W1.astype(f32).reshape(-1),
        W2.astype(f32).reshape(-1),
        Wl.astype(f32).reshape(-1), jnp.zeros((12,), f32),
        bl.astype(f32), jnp.zeros((15,), f32),
        b1.astype(f32), gamma1.astype(f32), beta1.astype(f32),
        b2.astype(f32), gamma2.astype(f32), beta2.astype(f32),
        b_ih1.astype(f32), b_hh1.astype(f32),
        b_ih2.astype(f32), b_hh2.astype(f32),
    ])

    mesh = plsc.VectorSubcoreMesh(core_axis_name="c", subcore_axis_name="s")
    out = pl.kernel(
        _body,
        out_type=jax.ShapeDtypeStruct((N, 16), f32),
        mesh=mesh,
        compiler_params=pltpu.CompilerParams(needs_layout_passes=False),
        scratch_types=[
            pltpu.VMEM((PTOT,), f32),
            pltpu.VMEM((2 * EP,), jnp.int32),
            pltpu.VMEM((96 * 128,), f32),
            pltpu.VMEM((NP,), f32),
            pltpu.VMEM((NP * NP,), f32),
            pltpu.VMEM((N * HID,), f32),
            pltpu.VMEM((N * HID,), f32),
            pltpu.VMEM((16,), f32),
            pltpu.SemaphoreType.DMA,
            pltpu.SemaphoreType.DMA,
        ],
    )(pack, edges, wcat)
    return out[:, :1]
